# Initial kernel scaffold; baseline (speedup 1.0000x reference)
import functools

import jax
import jax.numpy as jnp
import numpy as np
from jax import lax
from jax.experimental import pallas as pl
from jax.experimental.pallas import tpu as pltpu
from jax.experimental.pallas import tpu_sc as plsc

VOCAB = 1000000
DIM = 64
B = 4096
L = 200
N_SEG = 3

_NC = 2
_NS = 16
_NW = _NC * _NS
_N = B * L
_PW = _N // _NW
_SUB = 128
_NSUB = 4
_CH = _SUB * _NSUB
_NCHUNK = _PW // _CH


def _sinusoidal_pe(length, dim):
    pos = np.arange(length)[:, None].astype(np.float64)
    i = np.arange(dim)[None, :]
    angle_rates = 1.0 / np.power(10000.0, (2 * (i // 2)) / np.float64(dim))
    angles = pos * angle_rates
    pe = np.zeros((length, dim), dtype=np.float64)
    pe[:, 0::2] = np.sin(angles[:, 0::2])
    pe[:, 1::2] = np.cos(angles[:, 1::2])
    return pe.astype(np.float32)


_PE = _sinusoidal_pe(L, DIM)


def _combo_table(seg_table):
    def body(seg_ref, pe_ref, out_ref):
        out_ref[...] = seg_ref[...] + pe_ref[...]

    out = pl.pallas_call(
        body,
        out_shape=jax.ShapeDtypeStruct((N_SEG, L, DIM), jnp.float32),
    )(seg_table[:, None, :], jnp.asarray(_PE)[None, :, :])
    return out.reshape(N_SEG * L, DIM)


def _sc_lookup(seq2d, lab2d, token_table, combo):
    mesh = plsc.VectorSubcoreMesh(core_axis_name="c", subcore_axis_name="s")

    @functools.partial(
        pl.kernel,
        out_type=jax.ShapeDtypeStruct((_N, DIM), jnp.float32),
        mesh=mesh,
        scratch_types=[
            pltpu.VMEM((_NSUB, _SUB), jnp.int32),
            pltpu.VMEM((_NSUB, _SUB), jnp.int32),
            pltpu.VMEM((_NSUB, _SUB), jnp.int32),
            pltpu.VMEM((_NSUB, _SUB, DIM), jnp.float32),
            pltpu.VMEM((_NSUB, _SUB, DIM), jnp.float32),
            pltpu.SemaphoreType.DMA,
            pltpu.SemaphoreType.DMA,
        ],
    )
    def k(seq_hbm, lab_hbm, tok_hbm, combo_hbm, out_hbm,
          idx_v, lab_v, ci_v, rows_t, rows_a, sem_t, sem_a):
        wid = lax.axis_index("s") * _NC + lax.axis_index("c")
        lane = lax.iota(jnp.int32, 16)

        def chunk_body(kk, carry):
            r0 = wid * (_PW // _SUB) + kk * _NSUB
            pltpu.sync_copy(seq_hbm.at[pl.ds(r0, _NSUB)], idx_v)
            pltpu.sync_copy(lab_hbm.at[pl.ds(r0, _NSUB)], lab_v)
            base = kk * _CH
            for j in range(_NSUB):
                for c in range(_SUB // 16):
                    pos = lax.rem(base + j * _SUB + c * 16 + lane, L)
                    ci_v[j, pl.ds(c * 16, 16)] = lab_v[j, pl.ds(c * 16, 16)] * L + pos
            cps = []
            for j in range(_NSUB):
                cps.append(pltpu.async_copy(tok_hbm.at[idx_v.at[j]], rows_t.at[j], sem_t))
                cps.append(pltpu.async_copy(combo_hbm.at[ci_v.at[j]], rows_a.at[j], sem_a))
            for cp in cps:
                cp.wait()
            for j in range(_NSUB):
                def add_body(i, c2):
                    for c in range(DIM // 16):
                        rows_t[j, i, pl.ds(c * 16, 16)] = (
                            rows_t[j, i, pl.ds(c * 16, 16)]
                            + rows_a[j, i, pl.ds(c * 16, 16)])
                    return c2
                lax.fori_loop(0, _SUB, add_body, 0)
            row_out = wid * _PW + kk * _CH
            for j in range(_NSUB):
                pltpu.sync_copy(rows_t.at[j], out_hbm.at[pl.ds(row_out + j * _SUB, _SUB)])
            return carry

        lax.fori_loop(0, _NCHUNK, chunk_body, 0)

    return k(seq2d, lab2d, token_table, combo)


def kernel(seq, seg_label, token_table, seg_table):
    combo = _combo_table(seg_table)
    seq2d = seq.reshape(_N // _SUB, _SUB).astype(jnp.int32)
    lab2d = seg_label.reshape(_N // _SUB, _SUB).astype(jnp.int32)
    out = _sc_lookup(seq2d, lab2d, token_table, combo)
    return out.reshape(B, L, DIM)


# speedup vs baseline: 2.2939x; 2.2939x over previous
import functools

import jax
import jax.numpy as jnp
import numpy as np
from jax import lax
from jax.experimental import pallas as pl
from jax.experimental.pallas import tpu as pltpu
from jax.experimental.pallas import tpu_sc as plsc

VOCAB = 1000000
DIM = 64
B = 4096
L = 200
N_SEG = 3

_NC = 2
_NS = 16
_NW = _NC * _NS
_N = B * L
_PW = _N // _NW
_SUB = 128
_NSUB = 4
_CH = _SUB * _NSUB
_NCHUNK = _PW // _CH


def _sinusoidal_pe(length, dim):
    pos = np.arange(length)[:, None].astype(np.float64)
    i = np.arange(dim)[None, :]
    angle_rates = 1.0 / np.power(10000.0, (2 * (i // 2)) / np.float64(dim))
    angles = pos * angle_rates
    pe = np.zeros((length, dim), dtype=np.float64)
    pe[:, 0::2] = np.sin(angles[:, 0::2])
    pe[:, 1::2] = np.cos(angles[:, 1::2])
    return pe.astype(np.float32)


_PE = _sinusoidal_pe(L, DIM)


def _combo_table(seg_table):
    def body(seg_ref, pe_ref, out_ref):
        out_ref[...] = seg_ref[...] + pe_ref[...]

    out = pl.pallas_call(
        body,
        out_shape=jax.ShapeDtypeStruct((N_SEG, L, DIM), jnp.float32),
    )(seg_table[:, None, :], jnp.asarray(_PE)[None, :, :])
    return out.reshape(N_SEG * L, DIM)


def _sc_lookup(seq2d, lab2d, token_table, combo):
    mesh = plsc.VectorSubcoreMesh(core_axis_name="c", subcore_axis_name="s")

    @functools.partial(
        pl.kernel,
        out_type=jax.ShapeDtypeStruct((_N, DIM), jnp.float32),
        mesh=mesh,
        compiler_params=pltpu.CompilerParams(use_tc_tiling_on_sc=False),
        scratch_types=[
            pltpu.VMEM((_NSUB, _SUB), jnp.int32),
            pltpu.VMEM((_NSUB, _SUB), jnp.int32),
            pltpu.VMEM((_NSUB, _SUB), jnp.int32),
            pltpu.VMEM((_NSUB, _SUB, DIM), jnp.float32),
            pltpu.VMEM((_NSUB, _SUB, DIM), jnp.float32),
            pltpu.SemaphoreType.DMA,
            pltpu.SemaphoreType.DMA,
        ],
    )
    def k(seq_hbm, lab_hbm, tok_hbm, combo_hbm, out_hbm,
          idx_v, lab_v, ci_v, rows_t, rows_a, sem_t, sem_a):
        wid = lax.axis_index("s") * _NC + lax.axis_index("c")
        lane = lax.iota(jnp.int32, 16)

        def chunk_body(kk, carry):
            r0 = wid * (_PW // _SUB) + kk * _NSUB
            pltpu.sync_copy(seq_hbm.at[pl.ds(r0, _NSUB)], idx_v)
            pltpu.sync_copy(lab_hbm.at[pl.ds(r0, _NSUB)], lab_v)
            base = kk * _CH
            for j in range(_NSUB):
                for c in range(_SUB // 16):
                    pos = lax.rem(base + j * _SUB + c * 16 + lane, L)
                    ci_v[j, pl.ds(c * 16, 16)] = lab_v[j, pl.ds(c * 16, 16)] * L + pos
            cps = []
            for j in range(_NSUB):
                cps.append(pltpu.async_copy(tok_hbm.at[idx_v.at[j]], rows_t.at[j], sem_t))
                cps.append(pltpu.async_copy(combo_hbm.at[ci_v.at[j]], rows_a.at[j], sem_a))
            for cp in cps:
                cp.wait()
            for j in range(_NSUB):
                def add_body(i, c2):
                    for c in range(DIM // 16):
                        rows_t[j, i, pl.ds(c * 16, 16)] = (
                            rows_t[j, i, pl.ds(c * 16, 16)]
                            + rows_a[j, i, pl.ds(c * 16, 16)])
                    return c2
                lax.fori_loop(0, _SUB, add_body, 0)
            row_out = wid * _PW + kk * _CH
            for j in range(_NSUB):
                pltpu.sync_copy(rows_t.at[j], out_hbm.at[pl.ds(row_out + j * _SUB, _SUB)])
            return carry

        lax.fori_loop(0, _NCHUNK, chunk_body, 0)

    return k(seq2d, lab2d, token_table, combo)


def kernel(seq, seg_label, token_table, seg_table):
    combo = _combo_table(seg_table)
    seq2d = seq.reshape(_N // _SUB, _SUB).astype(jnp.int32)
    lab2d = seg_label.reshape(_N // _SUB, _SUB).astype(jnp.int32)
    out = _sc_lookup(seq2d, lab2d, token_table, combo)
    return out.reshape(B, L, DIM)


# speedup vs baseline: 2.3057x; 1.0052x over previous
import functools

import jax
import jax.numpy as jnp
import numpy as np
from jax import lax
from jax.experimental import pallas as pl
from jax.experimental.pallas import tpu as pltpu
from jax.experimental.pallas import tpu_sc as plsc

VOCAB = 1000000
DIM = 64
B = 4096
L = 200
N_SEG = 3

_NC = 2
_NS = 16
_NW = _NC * _NS
_N = B * L
_PW = _N // _NW
_SUB = 128
_NSUB = 4
_CH = _SUB * _NSUB
_NCHUNK = _PW // _CH


def _sinusoidal_pe(length, dim):
    pos = np.arange(length)[:, None].astype(np.float64)
    i = np.arange(dim)[None, :]
    angle_rates = 1.0 / np.power(10000.0, (2 * (i // 2)) / np.float64(dim))
    angles = pos * angle_rates
    pe = np.zeros((length, dim), dtype=np.float64)
    pe[:, 0::2] = np.sin(angles[:, 0::2])
    pe[:, 1::2] = np.cos(angles[:, 1::2])
    return pe.astype(np.float32)


_PE = _sinusoidal_pe(L, DIM)


def _combo_table(seg_table):
    def body(seg_ref, pe_ref, out_ref):
        out_ref[...] = seg_ref[...] + pe_ref[...]

    out = pl.pallas_call(
        body,
        out_shape=jax.ShapeDtypeStruct((N_SEG, L, DIM), jnp.float32),
    )(seg_table[:, None, :], jnp.asarray(_PE)[None, :, :])
    return out.reshape(N_SEG * L, DIM)


def _sc_lookup(seq2d, lab2d, token_table, combo):
    mesh = plsc.VectorSubcoreMesh(core_axis_name="c", subcore_axis_name="s")

    @functools.partial(
        pl.kernel,
        out_type=jax.ShapeDtypeStruct((_N, DIM), jnp.float32),
        mesh=mesh,
        compiler_params=pltpu.CompilerParams(use_tc_tiling_on_sc=False),
        scratch_types=[
            pltpu.VMEM((_NSUB, _SUB), jnp.int32),
            pltpu.VMEM((_NSUB, _SUB), jnp.int32),
            pltpu.VMEM((_NSUB, _SUB), jnp.int32),
            pltpu.VMEM((_NSUB, _SUB, DIM), jnp.float32),
            pltpu.VMEM((_NSUB, _SUB, DIM), jnp.float32),
            pltpu.SemaphoreType.DMA,
            pltpu.SemaphoreType.DMA,
        ],
    )
    def k(seq_hbm, lab_hbm, tok_hbm, combo_hbm, out_hbm,
          idx_v, lab_v, ci_v, rows_t, rows_a, sem_t, sem_a):
        wid = lax.axis_index("s") * _NC + lax.axis_index("c")
        lane = lax.iota(jnp.int32, 16)

        def chunk_body(kk, carry):
            r0 = wid * (_PW // _SUB) + kk * _NSUB
            pltpu.sync_copy(seq_hbm.at[pl.ds(r0, _NSUB)], idx_v)
            pltpu.sync_copy(lab_hbm.at[pl.ds(r0, _NSUB)], lab_v)
            base = kk * _CH
            for j in range(_NSUB):
                for c in range(_SUB // 16):
                    pos = lax.rem(base + j * _SUB + c * 16 + lane, L)
                    ci_v[j, pl.ds(c * 16, 16)] = lab_v[j, pl.ds(c * 16, 16)] * L + pos
            cps = []
            for j in range(_NSUB):
                cps.append(pltpu.async_copy(combo_hbm.at[ci_v.at[j]], rows_t.at[j], sem_a))
            for cp in cps:
                cp.wait()
            cps = []
            for j in range(_NSUB):
                cps.append(pltpu.async_copy(tok_hbm.at[idx_v.at[j]], rows_t.at[j], sem_t, add=True))
            for cp in cps:
                cp.wait()
            row_out = wid * _PW + kk * _CH
            for j in range(_NSUB):
                pltpu.sync_copy(rows_t.at[j], out_hbm.at[pl.ds(row_out + j * _SUB, _SUB)])
            return carry

        lax.fori_loop(0, _NCHUNK, chunk_body, 0)

    return k(seq2d, lab2d, token_table, combo)


def kernel(seq, seg_label, token_table, seg_table):
    combo = _combo_table(seg_table)
    seq2d = seq.reshape(_N // _SUB, _SUB).astype(jnp.int32)
    lab2d = seg_label.reshape(_N // _SUB, _SUB).astype(jnp.int32)
    out = _sc_lookup(seq2d, lab2d, token_table, combo)
    return out.reshape(B, L, DIM)
